# Initial kernel scaffold; baseline (speedup 1.0000x reference)
import jax
import jax.numpy as jnp
from jax.experimental import pallas as pl


def kernel(input):
    raise NotImplementedError("write your pallas kernel here")


# speedup vs baseline: 2670.9022x; 2670.9022x over previous
import functools

import jax
import jax.numpy as jnp
from jax import lax
from jax.experimental import pallas as pl
from jax.experimental.pallas import tpu as pltpu
from jax.experimental.pallas import tpu_sc as plsc

B, C, D, H, W = 2, 2, 4, 512, 512
BC = B * C
HW = H * W
DHW = D * HW
NX = BC * DHW
PROW = 48
HT = 8
N_ITERS = 5
BONUS = 10.0
MAX_SHIFT = 0.6
EPS = 1e-07

NCORES = 2
NSUB = 16
NW = NCORES * NSUB
BLK = 4096
NBLK = NX // (NW * BLK)
IDXCAP = 16384
NB = 128
NG = NB // 16


def _shl(a):
    return jnp.concatenate([a[..., :1], a[..., :-1]], axis=-1)


def _shr(a):
    return jnp.concatenate([a[..., 1:], a[..., -1:]], axis=-1)


def _mask_p_body(xp_hbm, mask_ref, p_ref, slab, sem):
    bc = pl.program_id(0)
    ht = pl.program_id(1)
    h0 = ht * HT
    pltpu.make_async_copy(
        xp_hbm.at[bc, :, pl.ds(h0, 16), :], slab, sem
    ).start()
    pltpu.make_async_copy(
        xp_hbm.at[bc, :, pl.ds(h0, 16), :], slab, sem
    ).wait()
    a = slab[...]
    aw = jnp.maximum(jnp.maximum(a, _shl(a)), _shr(a))
    ah = jnp.maximum(jnp.maximum(aw[:, 0:HT], aw[:, 1:HT + 1]), aw[:, 2:HT + 2])
    pooled = []
    for d in range(D):
        lo, hi = max(d - 1, 0), min(d + 1, D - 1)
        pooled.append(jnp.maximum(jnp.maximum(ah[lo], ah[d]), ah[hi]))
    pooled = jnp.stack(pooled, axis=0)
    center = a[:, 1:HT + 1, :]
    mask_ref[0] = (center == pooled).astype(jnp.int32)
    taps = []
    for d in range(D):
        for dh in (-1, 0, 1):
            rows = a[d, 1 + dh:1 + dh + HT, :]
            taps.append(_shl(rows))
            taps.append(rows)
            taps.append(_shr(rows))
    t = jnp.stack(taps, axis=2)
    t = jnp.concatenate(
        [t, jnp.zeros((HT, W, PROW - 36), jnp.float32)], axis=2)
    p_ref[0] = t


@functools.partial(jax.jit, static_argnames=("interpret",))
def _mask_and_p(xp, interpret=False):
    return pl.pallas_call(
        _mask_p_body,
        grid=(BC, H // HT),
        in_specs=[pl.BlockSpec(memory_space=pl.ANY)],
        out_specs=[
            pl.BlockSpec((1, D, HT, W), lambda bc, ht: (bc, 0, ht, 0)),
            pl.BlockSpec((1, HT, W, PROW), lambda bc, ht: (bc, ht, 0, 0)),
        ],
        out_shape=[
            jax.ShapeDtypeStruct((BC, D, H, W), jnp.int32),
            jax.ShapeDtypeStruct((BC, H, W, PROW), jnp.float32),
        ],
        scratch_shapes=[
            pltpu.VMEM((D, 16, W), jnp.float32),
            pltpu.SemaphoreType.DMA,
        ],
        interpret=interpret,
    )(xp)


def _iota16():
    return lax.iota(jnp.int32, 16)


def _sc_refine_body(mask_hbm, p_hbm, y_hbm, c_hbm, cnt_hbm,
                    mblk, idxbuf, rowbuf, dscbuf, patches,
                    curd, curh, curw, vld, sxb, syb, ssb, gdsb,
                    val1, idx1, cnt16, sem, sem2):
    wid = lax.axis_index("s") * NCORES + lax.axis_index("c")

    def blk_body(b, tot):
        base = pl.multiple_of((b * NW + wid) * BLK, BLK)
        pltpu.sync_copy(mask_hbm.at[pl.ds(base, BLK)], mblk)

        def grp(j, off):
            v = mblk[pl.ds(j * 16, 16)]
            m = jnp.logical_and(v > 0, off < IDXCAP - 16)
            ids = jnp.full((16,), base + j * 16, jnp.int32) + _iota16()
            cum = plsc.cumsum(m.astype(jnp.int32))
            plsc.store_scatter(idxbuf, [off + cum - 1], ids, mask=m)
            return off + cum[15]

        return lax.fori_loop(0, BLK // 16, grp, tot)

    total = lax.fori_loop(0, NBLK, blk_body, jnp.int32(0))

    first = idxbuf[pl.ds(0, 16)][0]
    padto = ((total + NB - 1) // NB) * NB
    g0 = (total // 16) * 16

    def padg(k, _):
        s = g0 + k * 16
        pos = jnp.full((16,), s, jnp.int32) + _iota16()
        cur = idxbuf[pl.ds(s, 16)]
        idxbuf[pl.ds(s, 16)] = jnp.where(
            pos < total, cur, jnp.full((16,), first, jnp.int32))
        return 0

    lax.fori_loop(0, (padto - g0) // 16, padg, 0)

    def batch(i, _):
        sb = i * NB

        def init_g(g, _c):
            sl = pl.ds(g * 16, 16)
            lin = idxbuf[pl.ds(sb + g * 16, 16)]
            curd[sl] = jnp.right_shift(lin, 18) & 3
            curh[sl] = jnp.right_shift(lin, 9) & (H - 1)
            curw[sl] = lin & (W - 1)
            vld[sl] = jnp.ones((16,), jnp.int32)
            z = jnp.zeros((16,), jnp.float32)
            sxb[sl] = z
            syb[sl] = z
            ssb[sl] = z
            gdsb[sl] = z
            return 0

        lax.fori_loop(0, NG, init_g, 0)

        def rows_g(g, _c):
            sl = pl.ds(g * 16, 16)
            lin = idxbuf[pl.ds(sb + g * 16, 16)]
            bc = jnp.right_shift(lin, 20)
            dsc = jnp.clip(curd[sl], 1, D - 2)
            hsc = jnp.clip(curh[sl], 1, H - 2)
            wsc = jnp.clip(curw[sl], 1, W - 2)
            rowbuf[sl] = bc * HW + hsc * W + wsc
            dscbuf[sl] = dsc
            return 0

        def solve_g(g, _c):
            sl = pl.ds(g * 16, 16)
            rows = jnp.full((16,), g * 16, jnp.int32) + _iota16()
            q0 = dscbuf[sl] * 9 + 4

            def tap(o):
                return plsc.load_gather(patches, [rows, q0 + o])

            c000 = tap(0)
            pxm = tap(-1)
            pxp = tap(1)
            pym = tap(-3)
            pyp = tap(3)
            psm = tap(-9)
            psp = tap(9)
            gx = 0.5 * (pxp - pxm)
            gy = 0.5 * (pyp - pym)
            gs = 0.5 * (psp - psm)
            dxx = pxp - 2.0 * c000 + pxm
            dyy = pyp - 2.0 * c000 + pym
            dss = psp - 2.0 * c000 + psm
            dxy = 0.25 * (tap(4) - tap(2) - tap(-2) + tap(-4))
            dxs = 0.25 * (tap(10) - tap(8) - tap(-8) + tap(-10))
            dys = 0.25 * (tap(12) - tap(6) - tap(-6) + tap(-12))
            r0, r1, r2 = -gx, -gy, -gs
            cf00 = dyy * dss - dys * dys
            cf01 = dxy * dss - dys * dxs
            cf02 = dxy * dys - dyy * dxs
            det = dxx * cf00 - dxy * cf01 + dxs * cf02
            solved = jnp.abs(det) > EPS
            safe_det = jnp.where(solved, det, jnp.ones_like(det))
            sx = (r0 * cf00 - dxy * (r1 * dss - dys * r2)
                  + dxs * (r1 * dys - dyy * r2)) / safe_det
            sy = (dxx * (r1 * dss - dys * r2) - r0 * cf01
                  + dxs * (dxy * r2 - r1 * dxs)) / safe_det
            ss = (dxx * (dyy * r2 - r1 * dys) - dxy * (dxy * r2 - r1 * dxs)
                  + r0 * cf02) / safe_det
            valid = jnp.logical_and(vld[sl] > 0, solved)
            vf = valid.astype(jnp.float32)
            sx = sx * vf
            sy = sy * vf
            ss = ss * vf
            sxb[sl] = jnp.where(valid, sx, sxb[sl])
            syb[sl] = jnp.where(valid, sy, syb[sl])
            ssb[sl] = jnp.where(valid, ss, ssb[sl])
            gdsb[sl] = jnp.where(valid, gx * sx + gy * sy + gs * ss,
                                 gdsb[sl])
            one = jnp.ones((16,), jnp.int32)
            zero = jnp.zeros((16,), jnp.int32)
            mpx = jnp.where(jnp.logical_and(valid, sx > MAX_SHIFT), one, zero)
            mnx = jnp.where(jnp.logical_and(valid, sx < -MAX_SHIFT), one, zero)
            new_w = curw[sl] + mpx - mnx
            valid = jnp.logical_and(
                valid, jnp.logical_and(new_w >= 1, new_w <= W - 2))
            curw[sl] = jnp.clip(new_w, 0, W - 1)
            mpy = jnp.where(jnp.logical_and(valid, sy > MAX_SHIFT), one, zero)
            mny = jnp.where(jnp.logical_and(valid, sy < -MAX_SHIFT), one, zero)
            new_h = curh[sl] + mpy - mny
            valid = jnp.logical_and(
                valid, jnp.logical_and(new_h >= 1, new_h <= H - 2))
            curh[sl] = jnp.clip(new_h, 0, H - 1)
            mps = jnp.where(jnp.logical_and(valid, ss > MAX_SHIFT), one, zero)
            mns = jnp.where(jnp.logical_and(valid, ss < -MAX_SHIFT), one, zero)
            new_d = curd[sl] + mps - mns
            valid = jnp.logical_and(
                valid, jnp.logical_and(new_d >= 1, new_d <= D - 2))
            curd[sl] = jnp.clip(new_d, 0, D - 1)
            vld[sl] = valid.astype(jnp.int32)
            return 0

        for _it in range(N_ITERS):
            lax.fori_loop(0, NG, rows_g, 0)
            pltpu.async_copy(p_hbm.at[rowbuf], patches, sem).wait()
            lax.fori_loop(0, NG, solve_g, 0)

        def fin_g(g, _c):
            sl = pl.ds(g * 16, 16)
            lin = idxbuf[pl.ds(sb + g * 16, 16)]
            bc = jnp.right_shift(lin, 20)
            dfin = jnp.clip(curd[sl], 1, D - 2)
            hfin = jnp.clip(curh[sl], 1, H - 2)
            wfin = jnp.clip(curw[sl], 1, W - 2)
            rowbuf[sl] = bc * HW + hfin * W + wfin
            dscbuf[sl] = dfin
            curd[sl] = dfin
            curh[sl] = hfin
            curw[sl] = wfin
            return 0

        lax.fori_loop(0, NG, fin_g, 0)
        pltpu.async_copy(p_hbm.at[rowbuf], patches, sem).wait()

        def yval_g(g, _c):
            sl = pl.ds(g * 16, 16)
            rows = jnp.full((16,), g * 16, jnp.int32) + _iota16()
            cfin = plsc.load_gather(patches, [rows, dscbuf[sl] * 9 + 4])
            val1[sl] = cfin + 0.5 * gdsb[sl] + BONUS
            idx1[sl] = idxbuf[pl.ds(sb + g * 16, 16)]
            return 0

        lax.fori_loop(0, NG, yval_g, 0)
        pltpu.async_copy(val1, y_hbm.at[idx1], sem2).wait()

        def cval_g(ch, posref, shiftref):
            def body(g, _c):
                sl = pl.ds(g * 16, 16)
                lin = idxbuf[pl.ds(sb + g * 16, 16)]
                bc = jnp.right_shift(lin, 20)
                voxel = lin & (DHW - 1)
                idx1[sl] = bc * (3 * DHW) + ch * DHW + voxel
                val1[sl] = posref[sl].astype(jnp.float32) + shiftref[sl]
                return 0
            lax.fori_loop(0, NG, body, 0)
            pltpu.async_copy(val1, c_hbm.at[idx1], sem2).wait()

        cval_g(0, curd, ssb)
        cval_g(1, curw, sxb)
        cval_g(2, curh, syb)
        return 0

    nb = padto // NB
    lax.fori_loop(0, nb, batch, 0)

    cnt16[...] = jnp.full((16,), total, jnp.int32)
    pltpu.sync_copy(cnt16, cnt_hbm.at[wid])


def _sc_refine(mask_flat, p2, y_ref, c_ref):
    mesh = plsc.VectorSubcoreMesh(core_axis_name="c", subcore_axis_name="s")
    return pl.kernel(
        _sc_refine_body,
        out_type=jax.ShapeDtypeStruct((NW, 16), jnp.int32),
        mesh=mesh,
        compiler_params=pltpu.CompilerParams(
            needs_layout_passes=False, use_tc_tiling_on_sc=False),
        scratch_types=[
            pltpu.VMEM((BLK,), jnp.int32),
            pltpu.VMEM((IDXCAP,), jnp.int32),
            pltpu.VMEM((NB,), jnp.int32),
            pltpu.VMEM((NB,), jnp.int32),
            pltpu.VMEM((NB, PROW), jnp.float32),
            pltpu.VMEM((NB,), jnp.int32),
            pltpu.VMEM((NB,), jnp.int32),
            pltpu.VMEM((NB,), jnp.int32),
            pltpu.VMEM((NB,), jnp.int32),
            pltpu.VMEM((NB,), jnp.float32),
            pltpu.VMEM((NB,), jnp.float32),
            pltpu.VMEM((NB,), jnp.float32),
            pltpu.VMEM((NB,), jnp.float32),
            pltpu.VMEM((NB,), jnp.float32),
            pltpu.VMEM((NB,), jnp.int32),
            pltpu.VMEM((16,), jnp.int32),
            pltpu.SemaphoreType.DMA,
            pltpu.SemaphoreType.DMA,
        ],
    )(mask_flat, p2, y_ref, c_ref)


def kernel(input):
    x = input
    x4 = x.reshape(BC, D, H, W)
    xp = jnp.pad(x4, ((0, 0), (0, 0), (1, 7), (0, 0)), mode="edge")
    mask, p = _mask_and_p(xp)
    dt = jnp.float32
    dg = jnp.broadcast_to(
        jnp.arange(D, dtype=dt)[None, :, None, None], (BC, D, H, W))
    wg = jnp.broadcast_to(
        jnp.arange(W, dtype=dt)[None, None, None, :], (BC, D, H, W))
    hg = jnp.broadcast_to(
        jnp.arange(H, dtype=dt)[None, None, :, None], (BC, D, H, W))
    c0 = jnp.stack([dg, wg, hg], axis=1)
    y_ref = jax.new_ref(x4.reshape(NX))
    c_ref = jax.new_ref(c0.reshape(BC * 3 * DHW))
    _sc_refine(mask.reshape(NX), p.reshape(BC * HW, PROW), y_ref, c_ref)
    y = y_ref[...].reshape(B, C, D, H, W)
    coords = c_ref[...].reshape(B, C, 3, D, H, W)
    return coords, y
